# baseline (speedup 1.0000x reference)
import functools

import jax
import jax.numpy as jnp
from jax.experimental import pallas as pl

_RADIUS = 0.1
_K = 16
_OMIT = 5
_BQ = 400
_KP = 5120


def _knn_body(cloth_ref, obt_ref, vt_ref, dists_ref, idx_ref, valid_ref,
              active_ref):
    q = cloth_ref[...]
    obt = obt_ref[...]
    vt = vt_ref[...]

    qk = jnp.dot(q.astype(jnp.bfloat16), obt.astype(jnp.bfloat16),
                 preferred_element_type=jnp.float32)
    q2 = q[:, 0:1] ** 2 + q[:, 1:2] ** 2 + q[:, 2:3] ** 2
    k2 = obt[0:1, :] ** 2 + obt[1:2, :] ** 2 + obt[2:3, :] ** 2
    d2 = jnp.maximum((q2 + k2) - 2.0 * qk, 0.0)

    lane = jax.lax.broadcasted_iota(jnp.int32, (_BQ, _KP), 1)
    bad = (vt == _OMIT).astype(jnp.int32)
    code = 2 * lane + bad
    big_i = jnp.int32(2 ** 30)
    inf_f = jnp.float32(jnp.inf)

    w = d2
    dist_cols = []
    idx_cols = []
    valid_cols = []
    for _ in range(_K):
        m = jnp.min(w, axis=1, keepdims=True)
        sel = jnp.min(jnp.where(w == m, code, big_i), axis=1,
                      keepdims=True)
        one_hot = code == sel
        w = jnp.where(one_hot, inf_f, w)
        dist = jnp.sqrt(m + 1e-12)
        dist_cols.append(dist)
        idx_cols.append(jax.lax.shift_right_logical(sel, 1))
        valid_cols.append(((dist <= _RADIUS) &
                           ((sel & 1) == 0)).astype(jnp.int32))

    dists_ref[...] = jnp.concatenate(dist_cols, axis=1)
    idx_ref[...] = jnp.concatenate(idx_cols, axis=1)
    valid_ref[...] = jnp.concatenate(valid_cols, axis=1)

    contrib_pos = (jnp.isinf(w) & (jnp.sqrt(d2 + 1e-12) <= _RADIUS) &
                   (vt != _OMIT))
    contrib = jnp.max(contrib_pos.astype(jnp.int32), axis=0,
                      keepdims=True)

    @pl.when(pl.program_id(0) == 0)
    def _init():
        active_ref[...] = contrib

    @pl.when(pl.program_id(0) > 0)
    def _acc():
        active_ref[...] = jnp.maximum(active_ref[...], contrib)


@functools.partial(jax.jit, static_argnames=())
def kernel(cloth_pos, obstacle_pos, obstacle_vertex_type):
    q_n = cloth_pos.shape[0]
    k_n = obstacle_pos.shape[0]
    pad = _KP - k_n
    obt = jnp.concatenate(
        [obstacle_pos, jnp.full((pad, 3), 1e4, jnp.float32)], axis=0).T
    vt = jnp.concatenate(
        [obstacle_vertex_type, jnp.full((pad,), _OMIT, jnp.int32)]
    ).reshape(1, _KP)

    grid = q_n // _BQ
    dists, idx, valid_i, active = pl.pallas_call(
        _knn_body,
        grid=(grid,),
        in_specs=[
            pl.BlockSpec((_BQ, 3), lambda i: (i, 0)),
            pl.BlockSpec((3, _KP), lambda i: (0, 0)),
            pl.BlockSpec((1, _KP), lambda i: (0, 0)),
        ],
        out_specs=[
            pl.BlockSpec((_BQ, _K), lambda i: (i, 0)),
            pl.BlockSpec((_BQ, _K), lambda i: (i, 0)),
            pl.BlockSpec((_BQ, _K), lambda i: (i, 0)),
            pl.BlockSpec((1, _KP), lambda i: (0, 0)),
        ],
        out_shape=[
            jax.ShapeDtypeStruct((q_n, _K), jnp.float32),
            jax.ShapeDtypeStruct((q_n, _K), jnp.int32),
            jax.ShapeDtypeStruct((q_n, _K), jnp.int32),
            jax.ShapeDtypeStruct((1, _KP), jnp.int32),
        ],
    )(cloth_pos, obt, vt)

    indices_from = jnp.broadcast_to(
        jnp.arange(q_n, dtype=jnp.int32)[:, None], idx.shape)
    edges_direct = jnp.stack([indices_from, idx], axis=0)
    edges_inverse = jnp.stack([idx, indices_from], axis=0)
    valid = valid_i.astype(jnp.bool_)
    obstacle_active_mask = active[0, :k_n] > 0
    return dists, edges_direct, edges_inverse, valid, obstacle_active_mask


# speedup vs baseline: 1.0649x; 1.0649x over previous
import functools

import jax
import jax.numpy as jnp
from jax.experimental import pallas as pl

_RADIUS = 0.1
_K = 16
_OMIT = 5
_BQ = 80
_KP = 5120


def _knn_body(cloth_ref, obt_ref, vt_ref, dists_ref, idx_ref, valid_ref,
              active_ref):
    q = cloth_ref[...]
    obt = obt_ref[...]
    vt = vt_ref[...]

    qk = jnp.dot(q.astype(jnp.bfloat16), obt.astype(jnp.bfloat16),
                 preferred_element_type=jnp.float32)
    q2 = q[:, 0:1] ** 2 + q[:, 1:2] ** 2 + q[:, 2:3] ** 2
    k2 = obt[0:1, :] ** 2 + obt[1:2, :] ** 2 + obt[2:3, :] ** 2
    d2 = jnp.maximum((q2 + k2) - 2.0 * qk, 0.0)

    lane = jax.lax.broadcasted_iota(jnp.int32, (_BQ, _KP), 1)
    bad = (vt == _OMIT).astype(jnp.int32)
    code = 2 * lane + bad
    big_i = jnp.int32(2 ** 30)
    inf_f = jnp.float32(jnp.inf)

    w = d2
    dist_cols = []
    idx_cols = []
    valid_cols = []
    for _ in range(_K):
        m = jnp.min(w, axis=1, keepdims=True)
        sel = jnp.min(jnp.where(w == m, code, big_i), axis=1,
                      keepdims=True)
        one_hot = code == sel
        w = jnp.where(one_hot, inf_f, w)
        dist = jnp.sqrt(m + 1e-12)
        dist_cols.append(dist)
        idx_cols.append(jax.lax.shift_right_logical(sel, 1))
        valid_cols.append(((dist <= _RADIUS) &
                           ((sel & 1) == 0)).astype(jnp.int32))

    dists_ref[...] = jnp.concatenate(dist_cols, axis=1)
    idx_ref[...] = jnp.concatenate(idx_cols, axis=1)
    valid_ref[...] = jnp.concatenate(valid_cols, axis=1)

    contrib_pos = (jnp.isinf(w) & (jnp.sqrt(d2 + 1e-12) <= _RADIUS) &
                   (vt != _OMIT))
    contrib = jnp.max(contrib_pos.astype(jnp.int32), axis=0,
                      keepdims=True)

    @pl.when(pl.program_id(0) == 0)
    def _init():
        active_ref[...] = contrib

    @pl.when(pl.program_id(0) > 0)
    def _acc():
        active_ref[...] = jnp.maximum(active_ref[...], contrib)


@functools.partial(jax.jit, static_argnames=())
def kernel(cloth_pos, obstacle_pos, obstacle_vertex_type):
    q_n = cloth_pos.shape[0]
    k_n = obstacle_pos.shape[0]
    pad = _KP - k_n
    obt = jnp.concatenate(
        [obstacle_pos, jnp.full((pad, 3), 1e4, jnp.float32)], axis=0).T
    vt = jnp.concatenate(
        [obstacle_vertex_type, jnp.full((pad,), _OMIT, jnp.int32)]
    ).reshape(1, _KP)

    grid = q_n // _BQ
    dists, idx, valid_i, active = pl.pallas_call(
        _knn_body,
        grid=(grid,),
        in_specs=[
            pl.BlockSpec((_BQ, 3), lambda i: (i, 0)),
            pl.BlockSpec((3, _KP), lambda i: (0, 0)),
            pl.BlockSpec((1, _KP), lambda i: (0, 0)),
        ],
        out_specs=[
            pl.BlockSpec((_BQ, _K), lambda i: (i, 0)),
            pl.BlockSpec((_BQ, _K), lambda i: (i, 0)),
            pl.BlockSpec((_BQ, _K), lambda i: (i, 0)),
            pl.BlockSpec((1, _KP), lambda i: (0, 0)),
        ],
        out_shape=[
            jax.ShapeDtypeStruct((q_n, _K), jnp.float32),
            jax.ShapeDtypeStruct((q_n, _K), jnp.int32),
            jax.ShapeDtypeStruct((q_n, _K), jnp.int32),
            jax.ShapeDtypeStruct((1, _KP), jnp.int32),
        ],
    )(cloth_pos, obt, vt)

    indices_from = jnp.broadcast_to(
        jnp.arange(q_n, dtype=jnp.int32)[:, None], idx.shape)
    edges_direct = jnp.stack([indices_from, idx], axis=0)
    edges_inverse = jnp.stack([idx, indices_from], axis=0)
    valid = valid_i.astype(jnp.bool_)
    obstacle_active_mask = active[0, :k_n] > 0
    return dists, edges_direct, edges_inverse, valid, obstacle_active_mask


# speedup vs baseline: 1.1628x; 1.0919x over previous
import functools

import jax
import jax.numpy as jnp
from jax.experimental import pallas as pl
from jax.experimental.pallas import tpu as pltpu

_RADIUS = 0.1
_K = 16
_OMIT = 5
_BQ = 200
_KP = 5120


def _knn_body(cloth_ref, obt_ref, vt_ref, dists_ref, idx_ref, valid_ref,
              active_ref):
    q = cloth_ref[...]
    obt = obt_ref[...]
    vt = vt_ref[...]

    qk = jnp.dot(q.astype(jnp.bfloat16), obt.astype(jnp.bfloat16),
                 preferred_element_type=jnp.float32)
    q2 = q[:, 0:1] ** 2 + q[:, 1:2] ** 2 + q[:, 2:3] ** 2
    k2 = obt[0:1, :] ** 2 + obt[1:2, :] ** 2 + obt[2:3, :] ** 2
    d2 = jnp.maximum((q2 + k2) - 2.0 * qk, 0.0)

    lane = jax.lax.broadcasted_iota(jnp.int32, (_BQ, _KP), 1)
    bad = (vt == _OMIT).astype(jnp.int32)
    code = 2 * lane + bad
    big_i = jnp.int32(2 ** 30)
    inf_f = jnp.float32(jnp.inf)

    w = d2
    dist_cols = []
    idx_cols = []
    valid_cols = []
    for _ in range(_K):
        m = jnp.min(w, axis=1, keepdims=True)
        sel = jnp.min(jnp.where(w == m, code, big_i), axis=1,
                      keepdims=True)
        one_hot = code == sel
        w = jnp.where(one_hot, inf_f, w)
        dist = jnp.sqrt(m + 1e-12)
        dist_cols.append(dist)
        idx_cols.append(jax.lax.shift_right_logical(sel, 1))
        valid_cols.append(((dist <= _RADIUS) &
                           ((sel & 1) == 0)).astype(jnp.int32))

    dists_ref[...] = jnp.concatenate(dist_cols, axis=1)
    idx_ref[...] = jnp.concatenate(idx_cols, axis=1)
    valid_ref[...] = jnp.concatenate(valid_cols, axis=1)

    contrib_pos = (jnp.isinf(w) & (jnp.sqrt(d2 + 1e-12) <= _RADIUS) &
                   (vt != _OMIT))
    contrib = jnp.max(contrib_pos.astype(jnp.int32), axis=0,
                      keepdims=True)
    active_ref[...] = contrib[None]


@functools.partial(jax.jit, static_argnames=())
def kernel(cloth_pos, obstacle_pos, obstacle_vertex_type):
    q_n = cloth_pos.shape[0]
    k_n = obstacle_pos.shape[0]
    pad = _KP - k_n
    obt = jnp.concatenate(
        [obstacle_pos, jnp.full((pad, 3), 1e4, jnp.float32)], axis=0).T
    vt = jnp.concatenate(
        [obstacle_vertex_type, jnp.full((pad,), _OMIT, jnp.int32)]
    ).reshape(1, _KP)

    grid = q_n // _BQ
    dists, idx, valid_i, active = pl.pallas_call(
        _knn_body,
        grid=(grid,),
        compiler_params=pltpu.CompilerParams(
            dimension_semantics=("parallel",)),
        in_specs=[
            pl.BlockSpec((_BQ, 3), lambda i: (i, 0)),
            pl.BlockSpec((3, _KP), lambda i: (0, 0)),
            pl.BlockSpec((1, _KP), lambda i: (0, 0)),
        ],
        out_specs=[
            pl.BlockSpec((_BQ, _K), lambda i: (i, 0)),
            pl.BlockSpec((_BQ, _K), lambda i: (i, 0)),
            pl.BlockSpec((_BQ, _K), lambda i: (i, 0)),
            pl.BlockSpec((1, 1, _KP), lambda i: (i, 0, 0)),
        ],
        out_shape=[
            jax.ShapeDtypeStruct((q_n, _K), jnp.float32),
            jax.ShapeDtypeStruct((q_n, _K), jnp.int32),
            jax.ShapeDtypeStruct((q_n, _K), jnp.int32),
            jax.ShapeDtypeStruct((grid, 1, _KP), jnp.int32),
        ],
    )(cloth_pos, obt, vt)

    indices_from = jnp.broadcast_to(
        jnp.arange(q_n, dtype=jnp.int32)[:, None], idx.shape)
    edges_direct = jnp.stack([indices_from, idx], axis=0)
    edges_inverse = jnp.stack([idx, indices_from], axis=0)
    valid = valid_i.astype(jnp.bool_)
    obstacle_active_mask = jnp.max(active[:, 0, :k_n], axis=0) > 0
    return dists, edges_direct, edges_inverse, valid, obstacle_active_mask


# speedup vs baseline: 1.2176x; 1.0471x over previous
import functools

import jax
import jax.numpy as jnp
from jax.experimental import pallas as pl
from jax.experimental.pallas import tpu as pltpu

_RADIUS = 0.1
_K = 16
_OMIT = 5
_BQ = 200
_NCH = 48
_KP = _NCH * 128
_IMAX = 0x7FFFFFFF


def _batcher16():
    n, pairs, p = 16, [], 1
    while p < n:
        k = p
        while k >= 1:
            for j in range(k % p, n - k, 2 * k):
                for i in range(min(k, n - j - k)):
                    if (i + j) // (2 * p) == (i + j + k) // (2 * p):
                        pairs.append((i + j, i + j + k))
            k //= 2
        p *= 2
    return pairs


def _bitonic16():
    pairs, d = [], 8
    while d >= 1:
        for i in range(16):
            if (i % (2 * d)) < d:
                pairs.append((i, i + d))
        d //= 2
    return pairs


_SORT16 = _batcher16()
_MERGE16 = _bitonic16()


def _sort16(v):
    for i, j in _SORT16:
        a, b = v[i], v[j]
        v[i] = jnp.minimum(a, b)
        v[j] = jnp.maximum(a, b)


def _knn_body(cloth_ref, obt_ref, meta_ref, dists_ref, idx_ref, valid_ref,
              active_ref):
    q = cloth_ref[...]
    obt = obt_ref[...]
    meta = meta_ref[...]

    qk = jnp.dot(q.astype(jnp.bfloat16), obt.astype(jnp.bfloat16),
                 preferred_element_type=jnp.float32)
    q2 = q[:, 0:1] ** 2 + q[:, 1:2] ** 2 + q[:, 2:3] ** 2
    k2 = obt[0:1, :] ** 2 + obt[1:2, :] ** 2 + obt[2:3, :] ** 2
    d2 = jnp.maximum((q2 + k2) - 2.0 * qk, 0.0)

    s = jnp.minimum(d2 * 0.5, 1.75)
    bits = jax.lax.bitcast_convert_type(s, jnp.int32)
    key = (bits & jnp.int32(-128)) | meta

    sl = [key[:, c * 128:(c + 1) * 128] for c in range(_NCH)]
    g0, g1, g2 = sl[0:16], sl[16:32], sl[32:48]
    _sort16(g0)
    _sort16(g1)
    _sort16(g2)
    h = [jnp.minimum(g0[i], g1[15 - i]) for i in range(16)]
    for i, j in _MERGE16:
        a, b = h[i], h[j]
        h[i] = jnp.minimum(a, b)
        h[j] = jnp.maximum(a, b)
    cand = jnp.concatenate(
        [jnp.minimum(h[i], g2[15 - i]) for i in range(16)], axis=1)

    lane16 = jax.lax.broadcasted_iota(jnp.int32, (_BQ, 16 * 128), 1) & 127
    mk_cols = []
    lane_cols = []
    for _ in range(_K):
        mk = jnp.min(cand, axis=1, keepdims=True)
        eq = cand == mk
        lane_sel = jnp.min(jnp.where(eq, lane16, jnp.int32(_IMAX)), axis=1,
                           keepdims=True)
        one_hot = eq & (lane16 == lane_sel)
        cand = jnp.where(one_hot, jnp.int32(_IMAX), cand)
        mk_cols.append(mk)
        lane_cols.append(lane_sel)

    mks = jnp.concatenate(mk_cols, axis=1)
    lanes = jnp.concatenate(lane_cols, axis=1)

    j16 = jax.lax.broadcasted_iota(jnp.int32, (_BQ, _K), 1)
    even_left = (j16 & 1) == 0
    edge = (j16 == 0) | (j16 == _K - 1)
    for ph in range(_K):
        rm1_m = jnp.concatenate([mks[:, 1:], mks[:, :1]], axis=1)
        rp1_m = jnp.concatenate([mks[:, _K - 1:], mks[:, :_K - 1]], axis=1)
        rm1_l = jnp.concatenate([lanes[:, 1:], lanes[:, :1]], axis=1)
        rp1_l = jnp.concatenate([lanes[:, _K - 1:], lanes[:, :_K - 1]], axis=1)
        if ph % 2 == 0:
            is_left = even_left
            pm = jnp.where(is_left, rm1_m, rp1_m)
            plv = jnp.where(is_left, rm1_l, rp1_l)
        else:
            pair_left = ~even_left
            pm = jnp.where(pair_left, rm1_m, rp1_m)
            plv = jnp.where(pair_left, rm1_l, rp1_l)
            pm = jnp.where(edge, mks, pm)
            plv = jnp.where(edge, lanes, plv)
            is_left = pair_left | edge
        m2 = mks >> 1
        pm2 = pm >> 1
        le = (m2 < pm2) | ((m2 == pm2) & (lanes <= plv))
        keep = le == is_left
        mks = jnp.where(keep, mks, pm)
        lanes = jnp.where(keep, lanes, plv)

    s_q = jax.lax.bitcast_convert_type(mks & jnp.int32(-128), jnp.float32)
    dists = jnp.sqrt(2.0 * s_q + 1e-12)
    idx = ((mks >> 1) & 63) * 128 + lanes
    valid = ((dists <= _RADIUS) & ((mks & 1) == 0)).astype(jnp.int32)

    dists_ref[...] = dists
    idx_ref[...] = idx
    valid_ref[...] = valid

    t = mk_cols[_K - 1]
    contrib_pos = ((key <= t) & (jnp.sqrt(d2 + 1e-12) <= _RADIUS) &
                   ((meta & 1) == 0))
    contrib = jnp.max(contrib_pos.astype(jnp.int32), axis=0,
                      keepdims=True)

    @pl.when(pl.program_id(0) == 0)
    def _init():
        active_ref[...] = contrib

    @pl.when(pl.program_id(0) > 0)
    def _acc():
        active_ref[...] = jnp.maximum(active_ref[...], contrib)


@functools.partial(jax.jit, static_argnames=())
def kernel(cloth_pos, obstacle_pos, obstacle_vertex_type):
    q_n = cloth_pos.shape[0]
    k_n = obstacle_pos.shape[0]
    pad = _KP - k_n
    obt = jnp.concatenate(
        [obstacle_pos, jnp.full((pad, 3), 1e4, jnp.float32)], axis=0).T
    vt = jnp.concatenate(
        [obstacle_vertex_type, jnp.full((pad,), _OMIT, jnp.int32)])
    chunk_id = jnp.arange(_KP, dtype=jnp.int32) // 128
    meta = ((chunk_id << 1) | (vt == _OMIT).astype(jnp.int32)).reshape(1, _KP)

    grid = q_n // _BQ
    dists, idx, valid_i, active = pl.pallas_call(
        _knn_body,
        grid=(grid,),
        in_specs=[
            pl.BlockSpec((_BQ, 3), lambda i: (i, 0)),
            pl.BlockSpec((3, _KP), lambda i: (0, 0)),
            pl.BlockSpec((1, _KP), lambda i: (0, 0)),
        ],
        out_specs=[
            pl.BlockSpec((_BQ, _K), lambda i: (i, 0)),
            pl.BlockSpec((_BQ, _K), lambda i: (i, 0)),
            pl.BlockSpec((_BQ, _K), lambda i: (i, 0)),
            pl.BlockSpec((1, _KP), lambda i: (0, 0)),
        ],
        out_shape=[
            jax.ShapeDtypeStruct((q_n, _K), jnp.float32),
            jax.ShapeDtypeStruct((q_n, _K), jnp.int32),
            jax.ShapeDtypeStruct((q_n, _K), jnp.int32),
            jax.ShapeDtypeStruct((1, _KP), jnp.int32),
        ],
    )(cloth_pos, obt, meta)

    indices_from = jnp.broadcast_to(
        jnp.arange(q_n, dtype=jnp.int32)[:, None], idx.shape)
    edges_direct = jnp.stack([indices_from, idx], axis=0)
    edges_inverse = jnp.stack([idx, indices_from], axis=0)
    valid = valid_i.astype(jnp.bool_)
    obstacle_active_mask = active[0, :k_n] > 0
    return dists, edges_direct, edges_inverse, valid, obstacle_active_mask


# speedup vs baseline: 1.4891x; 1.2230x over previous
import functools

import jax
import jax.numpy as jnp
import numpy as np
from jax.experimental import pallas as pl
from jax.experimental.pallas import tpu as pltpu

_RADIUS = 0.1
_K = 16
_OMIT = 5
_BQ = 200
_NCH = 48
_KP = _NCH * 128
_IMAX = 0x7FFFFFFF


def _rad2_bound():
    x = np.float32(0.01)
    while np.sqrt(np.nextafter(x, np.float32(1)) + np.float32(1e-12))             <= np.float32(0.1):
        x = np.nextafter(x, np.float32(1))
    while np.sqrt(x + np.float32(1e-12)) > np.float32(0.1):
        x = np.nextafter(x, np.float32(0))
    bits = int((x * np.float32(0.5)).view(np.int32))
    return (bits & -128) | 127


_TW = _rad2_bound()


def _batcher16():
    n, pairs, p = 16, [], 1
    while p < n:
        k = p
        while k >= 1:
            for j in range(k % p, n - k, 2 * k):
                for i in range(min(k, n - j - k)):
                    if (i + j) // (2 * p) == (i + j + k) // (2 * p):
                        pairs.append((i + j, i + j + k))
            k //= 2
        p *= 2
    return pairs


def _bitonic16():
    pairs, d = [], 8
    while d >= 1:
        for i in range(16):
            if (i % (2 * d)) < d:
                pairs.append((i, i + d))
        d //= 2
    return pairs


_SORT16 = _batcher16()
_MERGE16 = _bitonic16()


def _sort16(v):
    for i, j in _SORT16:
        a, b = v[i], v[j]
        v[i] = jnp.minimum(a, b)
        v[j] = jnp.maximum(a, b)


def _knn_body(cloth_ref, obt_ref, zmeta_ref, dists_ref, idx_ref, valid_ref,
              active_ref):
    q = cloth_ref[...]
    obt = obt_ref[...]
    zmeta = zmeta_ref[...]
    meta = zmeta & 127

    qk = jnp.dot(q.astype(jnp.bfloat16), obt.astype(jnp.bfloat16),
                 preferred_element_type=jnp.float32)
    q2 = q[:, 0:1] ** 2 + q[:, 1:2] ** 2 + q[:, 2:3] ** 2
    k2 = obt[0:1, :] ** 2 + obt[1:2, :] ** 2 + obt[2:3, :] ** 2
    d2 = jnp.maximum((q2 + k2) - 2.0 * qk, 0.0)

    s = jnp.minimum(d2 * 0.5, 1.75)
    bits = jax.lax.bitcast_convert_type(s, jnp.int32)
    key = (bits & jnp.int32(-128)) | meta
    key = jnp.where(bits == 0, zmeta, key)

    sl = [key[:, c * 128:(c + 1) * 128] for c in range(_NCH)]
    g0, g1, g2 = sl[0:16], sl[16:32], sl[32:48]
    _sort16(g0)
    _sort16(g1)
    _sort16(g2)
    h = [jnp.minimum(g0[i], g1[15 - i]) for i in range(16)]
    for i, j in _MERGE16:
        a, b = h[i], h[j]
        h[i] = jnp.minimum(a, b)
        h[j] = jnp.maximum(a, b)
    cand = jnp.concatenate(
        [jnp.minimum(h[i], g2[15 - i]) for i in range(16)], axis=1)

    lane16 = jax.lax.broadcasted_iota(jnp.int32, (_BQ, 16 * 128), 1) & 127
    mk_cols = []
    lane_cols = []
    for _ in range(_K):
        mk = jnp.min(cand, axis=1, keepdims=True)
        eq = cand == mk
        lane_sel = jnp.min(jnp.where(eq, lane16, jnp.int32(_IMAX)), axis=1,
                           keepdims=True)
        one_hot = eq & (lane16 == lane_sel)
        cand = jnp.where(one_hot, jnp.int32(_IMAX), cand)
        mk_cols.append(mk)
        lane_cols.append(lane_sel)

    mks = jnp.concatenate(mk_cols, axis=1)
    lanes = jnp.concatenate(lane_cols, axis=1)

    s_q = jax.lax.bitcast_convert_type(mks & jnp.int32(-128), jnp.float32)
    dists = jnp.sqrt(2.0 * s_q + 1e-12)
    idx = ((mks >> 1) & 63) * 128 + lanes
    valid = ((dists <= _RADIUS) & ((mks & 1) == 0)).astype(jnp.int32)

    dists_ref[...] = dists
    idx_ref[...] = idx
    valid_ref[...] = valid

    t = mk_cols[_K - 1]
    tt = jnp.minimum(t, jnp.int32(_TW))
    contrib_pos = (key <= tt) & ((meta & 1) == 0)
    contrib = jnp.max(contrib_pos.astype(jnp.int32), axis=0,
                      keepdims=True)

    @pl.when(pl.program_id(0) == 0)
    def _init():
        active_ref[...] = contrib

    @pl.when(pl.program_id(0) > 0)
    def _acc():
        active_ref[...] = jnp.maximum(active_ref[...], contrib)


@functools.partial(jax.jit, static_argnames=())
def kernel(cloth_pos, obstacle_pos, obstacle_vertex_type):
    q_n = cloth_pos.shape[0]
    k_n = obstacle_pos.shape[0]
    pad = _KP - k_n
    obt = jnp.concatenate(
        [obstacle_pos, jnp.full((pad, 3), 1e4, jnp.float32)], axis=0).T
    vt = jnp.concatenate(
        [obstacle_vertex_type, jnp.full((pad,), _OMIT, jnp.int32)])
    aridx = jnp.arange(_KP, dtype=jnp.int32)
    meta = ((aridx // 128) << 1) | (vt == _OMIT).astype(jnp.int32)
    zmeta = ((aridx << 7) | meta).reshape(1, _KP)

    grid = q_n // _BQ
    dists, idx, valid_i, active = pl.pallas_call(
        _knn_body,
        grid=(grid,),
        in_specs=[
            pl.BlockSpec((_BQ, 3), lambda i: (i, 0)),
            pl.BlockSpec((3, _KP), lambda i: (0, 0)),
            pl.BlockSpec((1, _KP), lambda i: (0, 0)),
        ],
        out_specs=[
            pl.BlockSpec((_BQ, _K), lambda i: (i, 0)),
            pl.BlockSpec((_BQ, _K), lambda i: (i, 0)),
            pl.BlockSpec((_BQ, _K), lambda i: (i, 0)),
            pl.BlockSpec((1, _KP), lambda i: (0, 0)),
        ],
        out_shape=[
            jax.ShapeDtypeStruct((q_n, _K), jnp.float32),
            jax.ShapeDtypeStruct((q_n, _K), jnp.int32),
            jax.ShapeDtypeStruct((q_n, _K), jnp.int32),
            jax.ShapeDtypeStruct((1, _KP), jnp.int32),
        ],
    )(cloth_pos, obt, zmeta)

    indices_from = jnp.broadcast_to(
        jnp.arange(q_n, dtype=jnp.int32)[:, None], idx.shape)
    edges_direct = jnp.stack([indices_from, idx], axis=0)
    edges_inverse = jnp.stack([idx, indices_from], axis=0)
    valid = valid_i.astype(jnp.bool_)
    obstacle_active_mask = active[0, :k_n] > 0
    return dists, edges_direct, edges_inverse, valid, obstacle_active_mask


# speedup vs baseline: 1.6880x; 1.1336x over previous
import functools

import jax
import jax.numpy as jnp
import numpy as np
from jax.experimental import pallas as pl
from jax.experimental.pallas import tpu as pltpu

_RADIUS = 0.1
_K = 16
_OMIT = 5
_BQ = 200
_NCH = 48
_KP = _NCH * 128
_IMAX = 0x7FFFFFFF


def _rad2_bound():
    x = np.float32(0.01)
    while np.sqrt(np.nextafter(x, np.float32(1)) + np.float32(1e-12))             <= np.float32(0.1):
        x = np.nextafter(x, np.float32(1))
    while np.sqrt(x + np.float32(1e-12)) > np.float32(0.1):
        x = np.nextafter(x, np.float32(0))
    bits = int((x * np.float32(0.5)).view(np.int32))
    return (bits & -128) | 127


_TW = _rad2_bound()


def _batcher16():
    n, pairs, p = 16, [], 1
    while p < n:
        k = p
        while k >= 1:
            for j in range(k % p, n - k, 2 * k):
                for i in range(min(k, n - j - k)):
                    if (i + j) // (2 * p) == (i + j + k) // (2 * p):
                        pairs.append((i + j, i + j + k))
            k //= 2
        p *= 2
    return pairs


def _bitonic16():
    pairs, d = [], 8
    while d >= 1:
        for i in range(16):
            if (i % (2 * d)) < d:
                pairs.append((i, i + d))
        d //= 2
    return pairs


_SORT16 = _batcher16()
_MERGE16 = _bitonic16()


def _sort16(v):
    for i, j in _SORT16:
        a, b = v[i], v[j]
        v[i] = jnp.minimum(a, b)
        v[j] = jnp.maximum(a, b)


def _knn_body(cloth_ref, obt_ref, zmeta_ref, dists_ref, idx_ref, valid_ref,
              active_ref):
    q = cloth_ref[...]
    obt = obt_ref[...]
    zmeta = zmeta_ref[...]
    meta = zmeta & 127

    qk = jnp.dot(q.astype(jnp.bfloat16), obt.astype(jnp.bfloat16),
                 preferred_element_type=jnp.float32)
    q2 = q[:, 0:1] ** 2 + q[:, 1:2] ** 2 + q[:, 2:3] ** 2
    k2 = obt[0:1, :] ** 2 + obt[1:2, :] ** 2 + obt[2:3, :] ** 2
    d2 = jnp.maximum((q2 + k2) - 2.0 * qk, 0.0)

    s = jnp.minimum(d2 * 0.5, 1.75)
    bits = jax.lax.bitcast_convert_type(s, jnp.int32)
    key = (bits & jnp.int32(-128)) | meta
    key = jnp.where(bits == 0, zmeta, key)

    sl = [key[:, c * 128:(c + 1) * 128] for c in range(_NCH)]
    g0, g1, g2 = sl[0:16], sl[16:32], sl[32:48]
    _sort16(g0)
    _sort16(g1)
    _sort16(g2)
    h = [jnp.minimum(g0[i], g1[15 - i]) for i in range(16)]
    for i, j in _MERGE16:
        a, b = h[i], h[j]
        h[i] = jnp.minimum(a, b)
        h[j] = jnp.maximum(a, b)
    cand = jnp.concatenate(
        [jnp.minimum(h[i], g2[15 - i]) for i in range(16)], axis=1)

    lane16 = jax.lax.broadcasted_iota(jnp.int32, (_BQ, 16 * 128), 1) & 127
    lane128 = jax.lax.broadcasted_iota(jnp.int32, (_BQ, 128), 1)
    mk_cols = []
    lane_cols = []
    for _ in range(_K):
        m128 = jnp.minimum(
            jnp.minimum(
                jnp.minimum(jnp.minimum(cand[:, 0:128], cand[:, 128:256]),
                            jnp.minimum(cand[:, 256:384], cand[:, 384:512])),
                jnp.minimum(jnp.minimum(cand[:, 512:640], cand[:, 640:768]),
                            jnp.minimum(cand[:, 768:896], cand[:, 896:1024]))),
            jnp.minimum(
                jnp.minimum(jnp.minimum(cand[:, 1024:1152], cand[:, 1152:1280]),
                            jnp.minimum(cand[:, 1280:1408], cand[:, 1408:1536])),
                jnp.minimum(jnp.minimum(cand[:, 1536:1664], cand[:, 1664:1792]),
                            jnp.minimum(cand[:, 1792:1920], cand[:, 1920:2048]))))
        mk = jnp.min(m128, axis=1, keepdims=True)
        eq = cand == mk
        lane_sel = jnp.min(jnp.where(m128 == mk, lane128, jnp.int32(_IMAX)),
                           axis=1, keepdims=True)
        one_hot = eq & (lane16 == lane_sel)
        cand = jnp.where(one_hot, jnp.int32(_IMAX), cand)
        mk_cols.append(mk)
        lane_cols.append(lane_sel)

    mks = jnp.concatenate(mk_cols, axis=1)
    lanes = jnp.concatenate(lane_cols, axis=1)

    s_q = jax.lax.bitcast_convert_type(mks & jnp.int32(-128), jnp.float32)
    dists = jnp.sqrt(2.0 * s_q + 1e-12)
    idx = ((mks >> 1) & 63) * 128 + lanes
    valid = ((dists <= _RADIUS) & ((mks & 1) == 0)).astype(jnp.int32)

    dists_ref[...] = dists
    idx_ref[...] = idx
    valid_ref[...] = valid

    t = mk_cols[_K - 1]
    tt = jnp.minimum(t, jnp.int32(_TW))
    contrib_pos = (key <= tt) & ((meta & 1) == 0)
    contrib = jnp.max(contrib_pos.astype(jnp.int32), axis=0,
                      keepdims=True)

    @pl.when(pl.program_id(0) == 0)
    def _init():
        active_ref[...] = contrib

    @pl.when(pl.program_id(0) > 0)
    def _acc():
        active_ref[...] = jnp.maximum(active_ref[...], contrib)


@functools.partial(jax.jit, static_argnames=())
def kernel(cloth_pos, obstacle_pos, obstacle_vertex_type):
    q_n = cloth_pos.shape[0]
    k_n = obstacle_pos.shape[0]
    pad = _KP - k_n
    obt = jnp.concatenate(
        [obstacle_pos, jnp.full((pad, 3), 1e4, jnp.float32)], axis=0).T
    vt = jnp.concatenate(
        [obstacle_vertex_type, jnp.full((pad,), _OMIT, jnp.int32)])
    aridx = jnp.arange(_KP, dtype=jnp.int32)
    meta = ((aridx // 128) << 1) | (vt == _OMIT).astype(jnp.int32)
    zmeta = ((aridx << 7) | meta).reshape(1, _KP)

    grid = q_n // _BQ
    dists, idx, valid_i, active = pl.pallas_call(
        _knn_body,
        grid=(grid,),
        in_specs=[
            pl.BlockSpec((_BQ, 3), lambda i: (i, 0)),
            pl.BlockSpec((3, _KP), lambda i: (0, 0)),
            pl.BlockSpec((1, _KP), lambda i: (0, 0)),
        ],
        out_specs=[
            pl.BlockSpec((_BQ, _K), lambda i: (i, 0)),
            pl.BlockSpec((_BQ, _K), lambda i: (i, 0)),
            pl.BlockSpec((_BQ, _K), lambda i: (i, 0)),
            pl.BlockSpec((1, _KP), lambda i: (0, 0)),
        ],
        out_shape=[
            jax.ShapeDtypeStruct((q_n, _K), jnp.float32),
            jax.ShapeDtypeStruct((q_n, _K), jnp.int32),
            jax.ShapeDtypeStruct((q_n, _K), jnp.int32),
            jax.ShapeDtypeStruct((1, _KP), jnp.int32),
        ],
    )(cloth_pos, obt, zmeta)

    indices_from = jnp.broadcast_to(
        jnp.arange(q_n, dtype=jnp.int32)[:, None], idx.shape)
    edges_direct = jnp.stack([indices_from, idx], axis=0)
    edges_inverse = jnp.stack([idx, indices_from], axis=0)
    valid = valid_i.astype(jnp.bool_)
    obstacle_active_mask = active[0, :k_n] > 0
    return dists, edges_direct, edges_inverse, valid, obstacle_active_mask


# speedup vs baseline: 2.0056x; 1.1881x over previous
import functools

import jax
import jax.numpy as jnp
import numpy as np
from jax.experimental import pallas as pl
from jax.experimental.pallas import tpu as pltpu

_RADIUS = 0.1
_K = 16
_OMIT = 5
_BQ = 400
_NCH = 48
_KP = _NCH * 128
_IMAX = 0x7FFFFFFF


def _rad2_bound():
    x = np.float32(0.01)
    while np.sqrt(np.nextafter(x, np.float32(1)) + np.float32(1e-12))             <= np.float32(0.1):
        x = np.nextafter(x, np.float32(1))
    while np.sqrt(x + np.float32(1e-12)) > np.float32(0.1):
        x = np.nextafter(x, np.float32(0))
    bits = int((x * np.float32(0.5)).view(np.int32))
    return (bits & -128) | 127


_TW = _rad2_bound()


def _batcher16():
    n, pairs, p = 16, [], 1
    while p < n:
        k = p
        while k >= 1:
            for j in range(k % p, n - k, 2 * k):
                for i in range(min(k, n - j - k)):
                    if (i + j) // (2 * p) == (i + j + k) // (2 * p):
                        pairs.append((i + j, i + j + k))
            k //= 2
        p *= 2
    return pairs


def _bitonic16():
    pairs, d = [], 8
    while d >= 1:
        for i in range(16):
            if (i % (2 * d)) < d:
                pairs.append((i, i + d))
        d //= 2
    return pairs


_SORT16 = _batcher16()
_MERGE16 = _bitonic16()


def _sort16(v):
    for i, j in _SORT16:
        a, b = v[i], v[j]
        v[i] = jnp.minimum(a, b)
        v[j] = jnp.maximum(a, b)


def _knn_body(cloth_ref, obt_ref, zmeta_ref, dists_ref, idx_ref, valid_ref,
              active_ref):
    q = cloth_ref[...]
    obt = obt_ref[...]
    zmeta = zmeta_ref[...]
    meta = zmeta & 127

    qk = jnp.dot(q.astype(jnp.bfloat16), obt.astype(jnp.bfloat16),
                 preferred_element_type=jnp.float32)
    q2 = q[:, 0:1] ** 2 + q[:, 1:2] ** 2 + q[:, 2:3] ** 2
    k2 = obt[0:1, :] ** 2 + obt[1:2, :] ** 2 + obt[2:3, :] ** 2
    d2 = jnp.maximum((q2 + k2) - 2.0 * qk, 0.0)

    s = jnp.minimum(d2 * 0.5, 1.75)
    bits = jax.lax.bitcast_convert_type(s, jnp.int32)
    key = (bits & jnp.int32(-128)) | meta
    key = jnp.where(bits == 0, zmeta, key)

    sl = [key[:, c * 128:(c + 1) * 128] for c in range(_NCH)]
    g0, g1, g2 = sl[0:16], sl[16:32], sl[32:48]
    _sort16(g0)
    _sort16(g1)
    _sort16(g2)
    h = [jnp.minimum(g0[i], g1[15 - i]) for i in range(16)]
    for i, j in _MERGE16:
        a, b = h[i], h[j]
        h[i] = jnp.minimum(a, b)
        h[j] = jnp.maximum(a, b)
    cand = jnp.concatenate(
        [jnp.minimum(h[i], g2[15 - i]) for i in range(16)], axis=1)

    lane16 = jax.lax.broadcasted_iota(jnp.int32, (_BQ, 16 * 128), 1) & 127
    lane128 = jax.lax.broadcasted_iota(jnp.int32, (_BQ, 128), 1)
    mk_cols = []
    lane_cols = []
    for _ in range(_K):
        m128 = jnp.minimum(
            jnp.minimum(
                jnp.minimum(jnp.minimum(cand[:, 0:128], cand[:, 128:256]),
                            jnp.minimum(cand[:, 256:384], cand[:, 384:512])),
                jnp.minimum(jnp.minimum(cand[:, 512:640], cand[:, 640:768]),
                            jnp.minimum(cand[:, 768:896], cand[:, 896:1024]))),
            jnp.minimum(
                jnp.minimum(jnp.minimum(cand[:, 1024:1152], cand[:, 1152:1280]),
                            jnp.minimum(cand[:, 1280:1408], cand[:, 1408:1536])),
                jnp.minimum(jnp.minimum(cand[:, 1536:1664], cand[:, 1664:1792]),
                            jnp.minimum(cand[:, 1792:1920], cand[:, 1920:2048]))))
        mk = jnp.min(m128, axis=1, keepdims=True)
        eq = cand == mk
        lane_sel = jnp.min(jnp.where(m128 == mk, lane128, jnp.int32(_IMAX)),
                           axis=1, keepdims=True)
        one_hot = eq & (lane16 == lane_sel)
        cand = jnp.where(one_hot, jnp.int32(_IMAX), cand)
        mk_cols.append(mk)
        lane_cols.append(lane_sel)

    mks = jnp.concatenate(mk_cols, axis=1)
    lanes = jnp.concatenate(lane_cols, axis=1)

    s_q = jax.lax.bitcast_convert_type(mks & jnp.int32(-128), jnp.float32)
    dists = jnp.sqrt(2.0 * s_q + 1e-12)
    idx = ((mks >> 1) & 63) * 128 + lanes
    valid = ((dists <= _RADIUS) & ((mks & 1) == 0)).astype(jnp.int32)

    dists_ref[...] = dists
    idx_ref[...] = idx
    valid_ref[...] = valid

    t = mk_cols[_K - 1]
    tt = jnp.minimum(t, jnp.int32(_TW))
    contrib_pos = (key <= tt) & ((meta & 1) == 0)
    contrib = jnp.max(contrib_pos.astype(jnp.int32), axis=0,
                      keepdims=True)

    @pl.when(pl.program_id(0) == 0)
    def _init():
        active_ref[...] = contrib

    @pl.when(pl.program_id(0) > 0)
    def _acc():
        active_ref[...] = jnp.maximum(active_ref[...], contrib)


@functools.partial(jax.jit, static_argnames=())
def kernel(cloth_pos, obstacle_pos, obstacle_vertex_type):
    q_n = cloth_pos.shape[0]
    k_n = obstacle_pos.shape[0]
    pad = _KP - k_n
    obt = jnp.concatenate(
        [obstacle_pos, jnp.full((pad, 3), 1e4, jnp.float32)], axis=0).T
    vt = jnp.concatenate(
        [obstacle_vertex_type, jnp.full((pad,), _OMIT, jnp.int32)])
    aridx = jnp.arange(_KP, dtype=jnp.int32)
    meta = ((aridx // 128) << 1) | (vt == _OMIT).astype(jnp.int32)
    zmeta = ((aridx << 7) | meta).reshape(1, _KP)

    grid = q_n // _BQ
    dists, idx, valid_i, active = pl.pallas_call(
        _knn_body,
        grid=(grid,),
        in_specs=[
            pl.BlockSpec((_BQ, 3), lambda i: (i, 0)),
            pl.BlockSpec((3, _KP), lambda i: (0, 0)),
            pl.BlockSpec((1, _KP), lambda i: (0, 0)),
        ],
        out_specs=[
            pl.BlockSpec((_BQ, _K), lambda i: (i, 0)),
            pl.BlockSpec((_BQ, _K), lambda i: (i, 0)),
            pl.BlockSpec((_BQ, _K), lambda i: (i, 0)),
            pl.BlockSpec((1, _KP), lambda i: (0, 0)),
        ],
        out_shape=[
            jax.ShapeDtypeStruct((q_n, _K), jnp.float32),
            jax.ShapeDtypeStruct((q_n, _K), jnp.int32),
            jax.ShapeDtypeStruct((q_n, _K), jnp.int32),
            jax.ShapeDtypeStruct((1, _KP), jnp.int32),
        ],
    )(cloth_pos, obt, zmeta)

    indices_from = jnp.broadcast_to(
        jnp.arange(q_n, dtype=jnp.int32)[:, None], idx.shape)
    edges_direct = jnp.stack([indices_from, idx], axis=0)
    edges_inverse = jnp.stack([idx, indices_from], axis=0)
    valid = valid_i.astype(jnp.bool_)
    obstacle_active_mask = active[0, :k_n] > 0
    return dists, edges_direct, edges_inverse, valid, obstacle_active_mask
